# baseline (speedup 1.0000x reference)
import jax
import jax.numpy as jnp
from jax import lax
from jax.experimental import pallas as pl
from jax.experimental.pallas import tpu as pltpu
from jax.experimental.pallas import tpu_sc as plsc

VOCAB = 1000000
EMBED_DIM = 64
BATCH = 16384

_INFO = plsc.get_sparse_core_info()
_NC = _INFO.num_cores
_NS = _INFO.num_subcores
_DPC = EMBED_DIM // _NC
_PPT = BATCH // _NS
_VMAIN = (VOCAB // 128) * 128
_VTAIL = VOCAB - _VMAIN
_SH = VOCAB + 64
_GCHUNK = 128


_STRIPE = 62464
_LAST = _VMAIN - 15 * _STRIPE


def _stage_row(emb_hbm, tail_hbm, d, sid, sh, sem):
    @pl.when(sid < 15)
    def _():
        pltpu.async_copy(emb_hbm.at[d, pl.ds(sid * _STRIPE, _STRIPE)],
                         sh.at[pl.ds(sid * _STRIPE, _STRIPE)], sem)

    @pl.when(sid == 15)
    def _():
        pltpu.async_copy(emb_hbm.at[d, pl.ds(15 * _STRIPE, _LAST)],
                         sh.at[pl.ds(15 * _STRIPE, _LAST)], sem)
        pltpu.async_copy(tail_hbm.at[d, pl.ds(0, 128)],
                         sh.at[pl.ds(_VMAIN, 128)], sem)


def _drain_row(emb_hbm, tail_hbm, d, sid, sh, sem):
    @pl.when(sid < 15)
    def _():
        pltpu.make_async_copy(
            emb_hbm.at[d, pl.ds(sid * _STRIPE, _STRIPE)],
            sh.at[pl.ds(sid * _STRIPE, _STRIPE)], sem).wait()

    @pl.when(sid == 15)
    def _():
        pltpu.make_async_copy(
            emb_hbm.at[d, pl.ds(15 * _STRIPE, _LAST)],
            sh.at[pl.ds(15 * _STRIPE, _LAST)], sem).wait()
        pltpu.make_async_copy(tail_hbm.at[d, pl.ds(0, 128)],
                              sh.at[pl.ds(_VMAIN, 128)], sem).wait()


def _sc_body(emb_hbm, tail_hbm, x_hbm, y_hbm, out_hbm,
             xi_v, yi_v, xvals_v, yvals_v, acc_v,
             sh0, sh1, sem0, sem1, gsem_x, gsem_y):
    cid = lax.axis_index("c")
    sid = lax.axis_index("s")
    d0 = cid * _DPC

    pltpu.sync_copy(x_hbm.at[pl.ds(sid * _PPT, _PPT)], xi_v)
    pltpu.sync_copy(y_hbm.at[pl.ds(sid * _PPT, _PPT)], yi_v)

    _stage_row(emb_hbm, tail_hbm, d0, sid, sh0, sem0)
    _stage_row(emb_hbm, tail_hbm, d0 + 1, sid, sh1, sem1)

    def gather_mac(sh, acc):
        copies = []
        for j in range(_PPT // _GCHUNK):
            s = pl.ds(j * _GCHUNK, _GCHUNK)
            copies.append(pltpu.async_copy(
                sh.at[xi_v.at[s]], xvals_v.at[s], gsem_x))
            copies.append(pltpu.async_copy(
                sh.at[yi_v.at[s]], yvals_v.at[s], gsem_y))
        for cp in copies:
            cp.wait()
        for j in range(_PPT // 16):
            s = pl.ds(j * 16, 16)
            acc = acc + xvals_v[s] * yvals_v[s]
        return acc

    def pair_body(k, acc):
        r0 = 2 * k
        _drain_row(emb_hbm, tail_hbm, d0 + r0, sid, sh0, sem0)
        plsc.subcore_barrier()
        acc = gather_mac(sh0, acc)
        plsc.subcore_barrier()

        @pl.when(k < _DPC // 2 - 1)
        def _():
            _stage_row(emb_hbm, tail_hbm, d0 + r0 + 2, sid, sh0, sem0)

        _drain_row(emb_hbm, tail_hbm, d0 + r0 + 1, sid, sh1, sem1)
        plsc.subcore_barrier()
        acc = gather_mac(sh1, acc)
        plsc.subcore_barrier()

        @pl.when(k < _DPC // 2 - 1)
        def _():
            _stage_row(emb_hbm, tail_hbm, d0 + r0 + 3, sid, sh1, sem1)

        return acc

    acc = lax.fori_loop(0, _DPC // 2, pair_body, jnp.zeros((16,), jnp.float32))
    acc_v[...] = acc
    pltpu.sync_copy(acc_v, out_hbm.at[sid * _NC + cid])


def _tc_body(p_ref, o_ref):
    s = jnp.sum(p_ref[...])
    o_ref[...] = jnp.full((1, 1),
                          jnp.minimum(s, 0.0) - jnp.log1p(jnp.exp(-jnp.abs(s))),
                          jnp.float32)


def kernel(x, y_true, embeddings):
    emb_t = embeddings.T
    tail = jnp.pad(emb_t[:, _VMAIN:], ((0, 0), (0, 128 - _VTAIL)))
    mesh = plsc.VectorSubcoreMesh(core_axis_name="c", subcore_axis_name="s")
    sc = pl.kernel(
        _sc_body,
        mesh=mesh,
        out_type=jax.ShapeDtypeStruct((_NC * _NS, 16), jnp.float32),
        scratch_types=[
            pltpu.VMEM((_PPT,), jnp.int32),
            pltpu.VMEM((_PPT,), jnp.int32),
            pltpu.VMEM((_PPT,), jnp.float32),
            pltpu.VMEM((_PPT,), jnp.float32),
            pltpu.VMEM((16,), jnp.float32),
            pltpu.VMEM_SHARED((_SH,), jnp.float32),
            pltpu.VMEM_SHARED((_SH,), jnp.float32),
            pltpu.SemaphoreType.DMA,
            pltpu.SemaphoreType.DMA,
            pltpu.SemaphoreType.DMA,
            pltpu.SemaphoreType.DMA,
        ],
    )
    partials = sc(emb_t, tail, x, y_true)
    return pl.pallas_call(
        _tc_body,
        out_shape=jax.ShapeDtypeStruct((1, 1), jnp.float32),
    )(partials)


# speedup vs baseline: 1.0196x; 1.0196x over previous
import jax
import jax.numpy as jnp
from jax import lax
from jax.experimental import pallas as pl
from jax.experimental.pallas import tpu as pltpu
from jax.experimental.pallas import tpu_sc as plsc

VOCAB = 1000000
EMBED_DIM = 64
BATCH = 16384

_INFO = plsc.get_sparse_core_info()
_NC = _INFO.num_cores
_NS = _INFO.num_subcores
_DPC = EMBED_DIM // _NC
_PPT = BATCH // _NS
_VMAIN = (VOCAB // 128) * 128
_VTAIL = VOCAB - _VMAIN
_SH = VOCAB + 64
_GCHUNK = 128


def _stage_row(emb_hbm, tail_hbm, d, sh, sem):
    pltpu.async_copy(emb_hbm.at[d, pl.ds(0, _VMAIN)], sh.at[pl.ds(0, _VMAIN)],
                     sem)
    pltpu.async_copy(tail_hbm.at[d, pl.ds(0, 128)],
                     sh.at[pl.ds(_VMAIN, 128)], sem)


def _drain_row(emb_hbm, tail_hbm, d, sh, sem):
    pltpu.make_async_copy(emb_hbm.at[d, pl.ds(0, _VMAIN)],
                          sh.at[pl.ds(0, _VMAIN)], sem).wait()
    pltpu.make_async_copy(tail_hbm.at[d, pl.ds(0, 128)],
                          sh.at[pl.ds(_VMAIN, 128)], sem).wait()


def _sc_body(emb_hbm, tail_hbm, x_hbm, y_hbm, out_hbm,
             xi_v, yi_v, xvals_v, yvals_v, acc_v,
             sh0, sh1, sem0, sem1, gsem_x, gsem_y):
    cid = lax.axis_index("c")
    sid = lax.axis_index("s")
    d0 = cid * _DPC

    pltpu.sync_copy(x_hbm.at[pl.ds(sid * _PPT, _PPT)], xi_v)
    pltpu.sync_copy(y_hbm.at[pl.ds(sid * _PPT, _PPT)], yi_v)

    @pl.when(sid == 0)
    def _():
        _stage_row(emb_hbm, tail_hbm, d0, sh0, sem0)
        _stage_row(emb_hbm, tail_hbm, d0 + 1, sh1, sem1)

    def gather_mac(sh, acc):
        copies = []
        for j in range(_PPT // _GCHUNK):
            s = pl.ds(j * _GCHUNK, _GCHUNK)
            copies.append(pltpu.async_copy(
                sh.at[xi_v.at[s]], xvals_v.at[s], gsem_x))
            copies.append(pltpu.async_copy(
                sh.at[yi_v.at[s]], yvals_v.at[s], gsem_y))
        for cp in copies:
            cp.wait()
        for j in range(_PPT // 16):
            s = pl.ds(j * 16, 16)
            acc = acc + xvals_v[s] * yvals_v[s]
        return acc

    def pair_body(k, acc):
        r0 = 2 * k
        @pl.when(sid == 0)
        def _():
            _drain_row(emb_hbm, tail_hbm, d0 + r0, sh0, sem0)
        plsc.subcore_barrier()
        acc = gather_mac(sh0, acc)
        plsc.subcore_barrier()

        @pl.when((sid == 0) & (k < _DPC // 2 - 1))
        def _():
            _stage_row(emb_hbm, tail_hbm, d0 + r0 + 2, sh0, sem0)

        @pl.when(sid == 0)
        def _():
            _drain_row(emb_hbm, tail_hbm, d0 + r0 + 1, sh1, sem1)
        plsc.subcore_barrier()
        acc = gather_mac(sh1, acc)
        plsc.subcore_barrier()

        @pl.when((sid == 0) & (k < _DPC // 2 - 1))
        def _():
            _stage_row(emb_hbm, tail_hbm, d0 + r0 + 3, sh1, sem1)

        return acc

    acc = lax.fori_loop(0, _DPC // 2, pair_body, jnp.zeros((16,), jnp.float32))
    acc_v[...] = acc
    pltpu.sync_copy(acc_v, out_hbm.at[sid * _NC + cid])


def _tc_body(p_ref, o_ref):
    s = jnp.sum(p_ref[...])
    o_ref[...] = jnp.full((1, 1),
                          jnp.minimum(s, 0.0) - jnp.log1p(jnp.exp(-jnp.abs(s))),
                          jnp.float32)


def kernel(x, y_true, embeddings):
    emb_t = embeddings.T
    tail = jnp.pad(emb_t[:, _VMAIN:], ((0, 0), (0, 128 - _VTAIL)))
    mesh = plsc.VectorSubcoreMesh(core_axis_name="c", subcore_axis_name="s")
    sc = pl.kernel(
        _sc_body,
        mesh=mesh,
        out_type=jax.ShapeDtypeStruct((_NC * _NS, 16), jnp.float32),
        scratch_types=[
            pltpu.VMEM((_PPT,), jnp.int32),
            pltpu.VMEM((_PPT,), jnp.int32),
            pltpu.VMEM((_PPT,), jnp.float32),
            pltpu.VMEM((_PPT,), jnp.float32),
            pltpu.VMEM((16,), jnp.float32),
            pltpu.VMEM_SHARED((_SH,), jnp.float32),
            pltpu.VMEM_SHARED((_SH,), jnp.float32),
            pltpu.SemaphoreType.DMA,
            pltpu.SemaphoreType.DMA,
            pltpu.SemaphoreType.DMA,
            pltpu.SemaphoreType.DMA,
        ],
    )
    partials = sc(emb_t, tail, x, y_true)
    return pl.pallas_call(
        _tc_body,
        out_shape=jax.ShapeDtypeStruct((1, 1), jnp.float32),
    )(partials)
